# Initial kernel scaffold; baseline (speedup 1.0000x reference)
import jax
import jax.numpy as jnp
from jax.experimental import pallas as pl


def kernel(x, edge_index, W_self0, W_neigh0, b0, W_self1, W_neigh1, b1, W_self2, W_neigh2, b2, gamma0, beta0, gamma1, beta1):
    raise NotImplementedError("write your pallas kernel here")


# speedup vs baseline: 4.7560x; 4.7560x over previous
import jax
import jax.numpy as jnp
from jax import lax
from jax.experimental import pallas as pl
from jax.experimental.pallas import tpu as pltpu
from jax.experimental.pallas import tpu_sc as plsc

N = 10000
E = 320000
D = 128
EPS = 1e-5
NC = 2
NS = 16
NW = NC * NS
EPW = E // NW
C = 80
NCHUNK = EPW // C
RPSP = 632
NPAD = NS * RPSP

_MESH = plsc.VectorSubcoreMesh(
    core_axis_name="c", subcore_axis_name="s",
    num_cores=NC, num_subcores=NS)


def _agg_body(h_hbm, src_hbm, dst_hbm, zrows_hbm,
              p_hbm, src_c, dst_c, rows_v, acc_sh, sem):
  cid = lax.axis_index("c")
  sid = lax.axis_index("s")
  wid = sid * NC + cid
  pltpu.sync_copy(zrows_hbm, acc_sh.at[pl.ds(sid * RPSP, RPSP)])
  plsc.subcore_barrier()
  base = wid * EPW

  def chunk(g, carry):
    off = base + g * C
    pltpu.sync_copy(src_hbm.at[pl.ds(off, C)], src_c)
    pltpu.sync_copy(dst_hbm.at[pl.ds(off, C)], dst_c)
    pltpu.async_copy(h_hbm.at[src_c], rows_v, sem).wait()
    pltpu.sync_copy(rows_v, acc_sh.at[dst_c], add=True)
    return carry

  lax.fori_loop(0, NCHUNK, chunk, 0)
  plsc.subcore_barrier()
  pltpu.sync_copy(acc_sh.at[pl.ds(sid * RPSP, RPSP)],
                  p_hbm.at[cid, pl.ds(sid * RPSP, RPSP)])


_agg = pl.kernel(
    _agg_body,
    out_type=jax.ShapeDtypeStruct((NC, NPAD, D), jnp.float32),
    mesh=_MESH,
    scratch_types=(
        pltpu.VMEM((C,), jnp.int32),
        pltpu.VMEM((C,), jnp.int32),
        pltpu.VMEM((C, D), jnp.float32),
        pltpu.VMEM_SHARED((NPAD, D), jnp.float32),
        pltpu.SemaphoreType.DMA,
    ))


def _deg_body(dst_hbm, ones_hbm, zdeg_hbm,
              degp_hbm, dst_c, ones_v, dacc_sh):
  cid = lax.axis_index("c")
  sid = lax.axis_index("s")
  wid = sid * NC + cid
  pltpu.sync_copy(zdeg_hbm, dacc_sh.at[pl.ds(sid * RPSP, RPSP)])
  pltpu.sync_copy(ones_hbm, ones_v)
  plsc.subcore_barrier()
  base = wid * EPW

  def chunk(g, carry):
    pltpu.sync_copy(dst_hbm.at[pl.ds(base + g * C, C)], dst_c)
    pltpu.sync_copy(ones_v, dacc_sh.at[dst_c], add=True)
    return carry

  lax.fori_loop(0, NCHUNK, chunk, 0)
  plsc.subcore_barrier()
  pltpu.sync_copy(dacc_sh.at[pl.ds(sid * RPSP, RPSP)],
                  degp_hbm.at[cid, pl.ds(sid * RPSP, RPSP)])


_deg = pl.kernel(
    _deg_body,
    out_type=jax.ShapeDtypeStruct((NC, NPAD, D), jnp.float32),
    mesh=_MESH,
    scratch_types=(
        pltpu.VMEM((C,), jnp.int32),
        pltpu.VMEM((C, D), jnp.float32),
        pltpu.VMEM_SHARED((NPAD, D), jnp.float32),
    ))


def _tc0_body(x_ref, p_ref, degp_ref, ws_ref, wn_ref, b_ref, ga_ref, be_ref,
              out_ref, rdeg_ref):
  deg = degp_ref[0, :N, :1] + degp_ref[1, :N, :1]
  rdeg = 1.0 / jnp.maximum(deg, 1.0)
  rdeg_ref[...] = rdeg
  mean = (p_ref[0, :N] + p_ref[1, :N]) * rdeg
  z = (jnp.dot(x_ref[...], ws_ref[...], preferred_element_type=jnp.float32)
       + jnp.dot(mean, wn_ref[...], preferred_element_type=jnp.float32)
       + b_ref[...])
  mu = jnp.mean(z, axis=0, keepdims=True)
  var = jnp.mean((z - mu) * (z - mu), axis=0, keepdims=True)
  zn = ga_ref[...] * (z - mu) * lax.rsqrt(var + EPS) + be_ref[...]
  out_ref[...] = jnp.maximum(zn, 0.0)


_tc0 = pl.pallas_call(
    _tc0_body,
    out_shape=(jax.ShapeDtypeStruct((N, D), jnp.float32),
               jax.ShapeDtypeStruct((N, 1), jnp.float32)))


def _tc_mid_body(h_ref, p_ref, rdeg_ref, ws_ref, wn_ref, b_ref, ga_ref,
                 be_ref, out_ref):
  mean = (p_ref[0, :N] + p_ref[1, :N]) * rdeg_ref[...]
  z = (jnp.dot(h_ref[...], ws_ref[...], preferred_element_type=jnp.float32)
       + jnp.dot(mean, wn_ref[...], preferred_element_type=jnp.float32)
       + b_ref[...])
  mu = jnp.mean(z, axis=0, keepdims=True)
  var = jnp.mean((z - mu) * (z - mu), axis=0, keepdims=True)
  zn = ga_ref[...] * (z - mu) * lax.rsqrt(var + EPS) + be_ref[...]
  out_ref[...] = jnp.maximum(zn, 0.0)


_tc_mid = pl.pallas_call(
    _tc_mid_body,
    out_shape=jax.ShapeDtypeStruct((N, D), jnp.float32))


def _tc_last_body(h_ref, p_ref, rdeg_ref, ws_ref, wn_ref, b_ref, out_ref):
  mean = (p_ref[0, :N] + p_ref[1, :N]) * rdeg_ref[...]
  out_ref[...] = (
      jnp.dot(h_ref[...], ws_ref[...], preferred_element_type=jnp.float32)
      + jnp.dot(mean, wn_ref[...], preferred_element_type=jnp.float32)
      + b_ref[...])


_tc_last = pl.pallas_call(
    _tc_last_body,
    out_shape=jax.ShapeDtypeStruct((N, D), jnp.float32))


def kernel(x, edge_index, W_self0, W_neigh0, b0, W_self1, W_neigh1, b1,
           W_self2, W_neigh2, b2, gamma0, beta0, gamma1, beta1):
  src = edge_index[0].astype(jnp.int32)
  dst = edge_index[1].astype(jnp.int32)
  zrows = jnp.zeros((RPSP, D), jnp.float32)
  onesrow = jnp.ones((C, D), jnp.float32)
  b0r, b1r, b2r = (b.reshape(1, D) for b in (b0, b1, b2))
  g0r, g1r = gamma0.reshape(1, D), gamma1.reshape(1, D)
  be0r, be1r = beta0.reshape(1, D), beta1.reshape(1, D)

  degp = _deg(dst, onesrow, zrows)
  p0 = _agg(x, src, dst, zrows)
  h1, rdeg = _tc0(x, p0, degp, W_self0, W_neigh0, b0r, g0r, be0r)
  p1 = _agg(h1, src, dst, zrows)
  h2 = _tc_mid(h1, p1, rdeg, W_self1, W_neigh1, b1r, g1r, be1r)
  p2 = _agg(h2, src, dst, zrows)
  out = _tc_last(h2, p2, rdeg, W_self2, W_neigh2, b2r)
  return out


# speedup vs baseline: 6.5363x; 1.3743x over previous
import jax
import jax.numpy as jnp
from jax import lax
from jax.experimental import pallas as pl
from jax.experimental.pallas import tpu as pltpu
from jax.experimental.pallas import tpu_sc as plsc

N = 10000
E = 320000
D = 128
EPS = 1e-5
NC = 2
NS = 16
NW = NC * NS
EPW = E // NW
C = 80
NCHUNK = EPW // C
RPSP = 632
NPAD = NS * RPSP

_MESH = plsc.VectorSubcoreMesh(
    core_axis_name="c", subcore_axis_name="s",
    num_cores=NC, num_subcores=NS)


def _agg_body(h_hbm, src_hbm, dst_hbm, zrows_hbm, p_hbm,
              sa, da, sb, db, rows_a, rows_b, acc_sh,
              sem_ia, sem_ib, sem_ga, sem_gb):
  cid = lax.axis_index("c")
  sid = lax.axis_index("s")
  wid = sid * NC + cid
  pltpu.sync_copy(zrows_hbm, acc_sh.at[pl.ds(sid * RPSP, RPSP)])
  plsc.subcore_barrier()
  base = wid * EPW

  def idx_issue(g, s_buf, d_buf, sem):
    off = base + g * C
    pltpu.async_copy(src_hbm.at[pl.ds(off, C)], s_buf, sem)
    pltpu.async_copy(dst_hbm.at[pl.ds(off, C)], d_buf, sem)

  def idx_wait(s_buf, d_buf, sem):
    pltpu.make_async_copy(src_hbm.at[pl.ds(0, C)], s_buf, sem).wait()
    pltpu.make_async_copy(dst_hbm.at[pl.ds(0, C)], d_buf, sem).wait()

  idx_issue(0, sa, da, sem_ia)
  idx_issue(1, sb, db, sem_ib)
  idx_wait(sa, da, sem_ia)
  pltpu.async_copy(h_hbm.at[sa], rows_a, sem_ga)

  def phase(g, s_x, d_x, rows_x, sem_ix, sem_iy, sem_gx, s_y, d_y, rows_y,
            sem_gy):
    pltpu.make_async_copy(h_hbm.at[s_x], rows_x, sem_gx).wait()
    pltpu.sync_copy(rows_x, acc_sh.at[d_x], add=True)

    @pl.when(g + 2 < NCHUNK)
    def _():
      idx_issue(g + 2, s_x, d_x, sem_ix)

    idx_wait(s_y, d_y, sem_iy)
    pltpu.async_copy(h_hbm.at[s_y], rows_y, sem_gy)

  def pair(g2, carry):
    ga = 2 * g2
    phase(ga, sa, da, rows_a, sem_ia, sem_ib, sem_ga, sb, db, rows_b, sem_gb)
    phase(ga + 1, sb, db, rows_b, sem_ib, sem_ia, sem_gb, sa, da, rows_a,
          sem_ga)
    return carry

  lax.fori_loop(0, (NCHUNK - 1) // 2, pair, 0)
  pltpu.make_async_copy(h_hbm.at[sa], rows_a, sem_ga).wait()
  pltpu.sync_copy(rows_a, acc_sh.at[da], add=True)
  plsc.subcore_barrier()
  pltpu.sync_copy(acc_sh.at[pl.ds(sid * RPSP, RPSP)],
                  p_hbm.at[cid, pl.ds(sid * RPSP, RPSP)])


_agg = pl.kernel(
    _agg_body,
    out_type=jax.ShapeDtypeStruct((NC, NPAD, D), jnp.float32),
    mesh=_MESH,
    scratch_types=(
        pltpu.VMEM((C,), jnp.int32),
        pltpu.VMEM((C,), jnp.int32),
        pltpu.VMEM((C,), jnp.int32),
        pltpu.VMEM((C,), jnp.int32),
        pltpu.VMEM((C, D), jnp.float32),
        pltpu.VMEM((C, D), jnp.float32),
        pltpu.VMEM_SHARED((NPAD, D), jnp.float32),
        pltpu.SemaphoreType.DMA,
        pltpu.SemaphoreType.DMA,
        pltpu.SemaphoreType.DMA,
        pltpu.SemaphoreType.DMA,
    ))


def _deg_body(dst_hbm, ones_hbm, zdeg_hbm,
              degp_hbm, dst_c, ones_v, dacc_sh):
  cid = lax.axis_index("c")
  sid = lax.axis_index("s")
  wid = sid * NC + cid
  pltpu.sync_copy(zdeg_hbm, dacc_sh.at[pl.ds(sid * RPSP, RPSP)])
  pltpu.sync_copy(ones_hbm, ones_v)
  plsc.subcore_barrier()
  base = wid * EPW

  def chunk(g, carry):
    pltpu.sync_copy(dst_hbm.at[pl.ds(base + g * C, C)], dst_c)
    pltpu.sync_copy(ones_v, dacc_sh.at[dst_c], add=True)
    return carry

  lax.fori_loop(0, NCHUNK, chunk, 0)
  plsc.subcore_barrier()
  pltpu.sync_copy(dacc_sh.at[pl.ds(sid * RPSP, RPSP)],
                  degp_hbm.at[cid, pl.ds(sid * RPSP, RPSP)])


_deg = pl.kernel(
    _deg_body,
    out_type=jax.ShapeDtypeStruct((NC, NPAD, D), jnp.float32),
    mesh=_MESH,
    scratch_types=(
        pltpu.VMEM((C,), jnp.int32),
        pltpu.VMEM((C, D), jnp.float32),
        pltpu.VMEM_SHARED((NPAD, D), jnp.float32),
    ))


def _tc0_body(x_ref, p_ref, degp_ref, ws_ref, wn_ref, b_ref, ga_ref, be_ref,
              out_ref, rdeg_ref):
  deg = degp_ref[0, :N, :1] + degp_ref[1, :N, :1]
  rdeg = 1.0 / jnp.maximum(deg, 1.0)
  rdeg_ref[...] = rdeg
  mean = (p_ref[0, :N] + p_ref[1, :N]) * rdeg
  z = (jnp.dot(x_ref[...], ws_ref[...], preferred_element_type=jnp.float32)
       + jnp.dot(mean, wn_ref[...], preferred_element_type=jnp.float32)
       + b_ref[...])
  mu = jnp.mean(z, axis=0, keepdims=True)
  var = jnp.mean((z - mu) * (z - mu), axis=0, keepdims=True)
  zn = ga_ref[...] * (z - mu) * lax.rsqrt(var + EPS) + be_ref[...]
  out_ref[...] = jnp.maximum(zn, 0.0)


_tc0 = pl.pallas_call(
    _tc0_body,
    out_shape=(jax.ShapeDtypeStruct((N, D), jnp.float32),
               jax.ShapeDtypeStruct((N, 1), jnp.float32)))


def _tc_mid_body(h_ref, p_ref, rdeg_ref, ws_ref, wn_ref, b_ref, ga_ref,
                 be_ref, out_ref):
  mean = (p_ref[0, :N] + p_ref[1, :N]) * rdeg_ref[...]
  z = (jnp.dot(h_ref[...], ws_ref[...], preferred_element_type=jnp.float32)
       + jnp.dot(mean, wn_ref[...], preferred_element_type=jnp.float32)
       + b_ref[...])
  mu = jnp.mean(z, axis=0, keepdims=True)
  var = jnp.mean((z - mu) * (z - mu), axis=0, keepdims=True)
  zn = ga_ref[...] * (z - mu) * lax.rsqrt(var + EPS) + be_ref[...]
  out_ref[...] = jnp.maximum(zn, 0.0)


_tc_mid = pl.pallas_call(
    _tc_mid_body,
    out_shape=jax.ShapeDtypeStruct((N, D), jnp.float32))


def _tc_last_body(h_ref, p_ref, rdeg_ref, ws_ref, wn_ref, b_ref, out_ref):
  mean = (p_ref[0, :N] + p_ref[1, :N]) * rdeg_ref[...]
  out_ref[...] = (
      jnp.dot(h_ref[...], ws_ref[...], preferred_element_type=jnp.float32)
      + jnp.dot(mean, wn_ref[...], preferred_element_type=jnp.float32)
      + b_ref[...])


_tc_last = pl.pallas_call(
    _tc_last_body,
    out_shape=jax.ShapeDtypeStruct((N, D), jnp.float32))


def kernel(x, edge_index, W_self0, W_neigh0, b0, W_self1, W_neigh1, b1,
           W_self2, W_neigh2, b2, gamma0, beta0, gamma1, beta1):
  src = edge_index[0].astype(jnp.int32)
  dst = edge_index[1].astype(jnp.int32)
  zrows = jnp.zeros((RPSP, D), jnp.float32)
  onesrow = jnp.ones((C, D), jnp.float32)
  b0r, b1r, b2r = (b.reshape(1, D) for b in (b0, b1, b2))
  g0r, g1r = gamma0.reshape(1, D), gamma1.reshape(1, D)
  be0r, be1r = beta0.reshape(1, D), beta1.reshape(1, D)

  degp = _deg(dst, onesrow, zrows)
  p0 = _agg(x, src, dst, zrows)
  h1, rdeg = _tc0(x, p0, degp, W_self0, W_neigh0, b0r, g0r, be0r)
  p1 = _agg(h1, src, dst, zrows)
  h2 = _tc_mid(h1, p1, rdeg, W_self1, W_neigh1, b1r, g1r, be1r)
  p2 = _agg(h2, src, dst, zrows)
  out = _tc_last(h2, p2, rdeg, W_self2, W_neigh2, b2r)
  return out


# speedup vs baseline: 8.0153x; 1.2263x over previous
import jax
import jax.numpy as jnp
from jax import lax
from jax.experimental import pallas as pl
from jax.experimental.pallas import tpu as pltpu
from jax.experimental.pallas import tpu_sc as plsc

N = 10000
E = 320000
D = 128
EPS = 1e-5
NC = 2
NS = 16
NW = NC * NS
EPW = E // NW
C = 80
NCHUNK = EPW // C
RPSP = 632
NPAD = NS * RPSP

_MESH = plsc.VectorSubcoreMesh(
    core_axis_name="c", subcore_axis_name="s",
    num_cores=NC, num_subcores=NS)


def _agg_body(h_hbm, src_hbm, dst_hbm, zrows_hbm, p_hbm,
              sa, da, sb, db, rows_a, rows_b, acc_sh,
              sem_ia, sem_ib, sem_ga, sem_gb):
  cid = lax.axis_index("c")
  sid = lax.axis_index("s")
  wid = sid * NC + cid
  pltpu.sync_copy(zrows_hbm, acc_sh.at[pl.ds(sid * RPSP, RPSP)])
  plsc.subcore_barrier()
  base = wid * EPW

  def idx_issue(g, s_buf, d_buf, sem):
    off = base + g * C
    pltpu.async_copy(src_hbm.at[pl.ds(off, C)], s_buf, sem)
    pltpu.async_copy(dst_hbm.at[pl.ds(off, C)], d_buf, sem)

  def idx_wait(s_buf, d_buf, sem):
    pltpu.make_async_copy(src_hbm.at[pl.ds(0, C)], s_buf, sem).wait()
    pltpu.make_async_copy(dst_hbm.at[pl.ds(0, C)], d_buf, sem).wait()

  idx_issue(0, sa, da, sem_ia)
  idx_issue(1, sb, db, sem_ib)
  idx_wait(sa, da, sem_ia)
  pltpu.async_copy(h_hbm.at[sa], rows_a, sem_ga)

  def phase(g, s_x, d_x, rows_x, sem_ix, sem_iy, sem_gx, s_y, d_y, rows_y,
            sem_gy):
    pltpu.make_async_copy(h_hbm.at[s_x], rows_x, sem_gx).wait()
    idx_wait(s_y, d_y, sem_iy)
    pltpu.async_copy(h_hbm.at[s_y], rows_y, sem_gy)
    pltpu.sync_copy(rows_x, acc_sh.at[d_x], add=True)
    @pl.when(g + 2 < NCHUNK)
    def _():
      idx_issue(g + 2, s_x, d_x, sem_ix)

  def pair(g2, carry):
    ga = 2 * g2
    phase(ga, sa, da, rows_a, sem_ia, sem_ib, sem_ga, sb, db, rows_b, sem_gb)
    phase(ga + 1, sb, db, rows_b, sem_ib, sem_ia, sem_gb, sa, da, rows_a,
          sem_ga)
    return carry

  lax.fori_loop(0, (NCHUNK - 1) // 2, pair, 0)
  pltpu.make_async_copy(h_hbm.at[sa], rows_a, sem_ga).wait()
  pltpu.sync_copy(rows_a, acc_sh.at[da], add=True)
  plsc.subcore_barrier()
  pltpu.sync_copy(acc_sh.at[pl.ds(sid * RPSP, RPSP)],
                  p_hbm.at[cid, pl.ds(sid * RPSP, RPSP)])


_agg = pl.kernel(
    _agg_body,
    out_type=jax.ShapeDtypeStruct((NC, NPAD, D), jnp.float32),
    mesh=_MESH,
    scratch_types=(
        pltpu.VMEM((C,), jnp.int32),
        pltpu.VMEM((C,), jnp.int32),
        pltpu.VMEM((C,), jnp.int32),
        pltpu.VMEM((C,), jnp.int32),
        pltpu.VMEM((C, D), jnp.float32),
        pltpu.VMEM((C, D), jnp.float32),
        pltpu.VMEM_SHARED((NPAD, D), jnp.float32),
        pltpu.SemaphoreType.DMA,
        pltpu.SemaphoreType.DMA,
        pltpu.SemaphoreType.DMA,
        pltpu.SemaphoreType.DMA,
    ))


def _deg_body(dst_hbm, ones_hbm, zdeg_hbm,
              degp_hbm, dst_c, ones_v, dacc_sh):
  cid = lax.axis_index("c")
  sid = lax.axis_index("s")
  wid = sid * NC + cid
  pltpu.sync_copy(zdeg_hbm, dacc_sh.at[pl.ds(sid * RPSP, RPSP)])
  pltpu.sync_copy(ones_hbm, ones_v)
  plsc.subcore_barrier()
  base = wid * EPW

  def chunk(g, carry):
    pltpu.sync_copy(dst_hbm.at[pl.ds(base + g * C, C)], dst_c)
    pltpu.sync_copy(ones_v, dacc_sh.at[dst_c], add=True)
    return carry

  lax.fori_loop(0, NCHUNK, chunk, 0)
  plsc.subcore_barrier()
  pltpu.sync_copy(dacc_sh.at[pl.ds(sid * RPSP, RPSP)],
                  degp_hbm.at[cid, pl.ds(sid * RPSP, RPSP)])


_deg = pl.kernel(
    _deg_body,
    out_type=jax.ShapeDtypeStruct((NC, NPAD, D), jnp.float32),
    mesh=_MESH,
    scratch_types=(
        pltpu.VMEM((C,), jnp.int32),
        pltpu.VMEM((C, D), jnp.float32),
        pltpu.VMEM_SHARED((NPAD, D), jnp.float32),
    ))


def _tc0_body(x_ref, p_ref, degp_ref, ws_ref, wn_ref, b_ref, ga_ref, be_ref,
              out_ref, rdeg_ref):
  deg = degp_ref[0, :N, :1] + degp_ref[1, :N, :1]
  rdeg = 1.0 / jnp.maximum(deg, 1.0)
  rdeg_ref[...] = rdeg
  mean = (p_ref[0, :N] + p_ref[1, :N]) * rdeg
  z = (jnp.dot(x_ref[...], ws_ref[...], preferred_element_type=jnp.float32)
       + jnp.dot(mean, wn_ref[...], preferred_element_type=jnp.float32)
       + b_ref[...])
  mu = jnp.mean(z, axis=0, keepdims=True)
  var = jnp.mean((z - mu) * (z - mu), axis=0, keepdims=True)
  zn = ga_ref[...] * (z - mu) * lax.rsqrt(var + EPS) + be_ref[...]
  out_ref[...] = jnp.maximum(zn, 0.0)


_tc0 = pl.pallas_call(
    _tc0_body,
    out_shape=(jax.ShapeDtypeStruct((N, D), jnp.float32),
               jax.ShapeDtypeStruct((N, 1), jnp.float32)))


def _tc_mid_body(h_ref, p_ref, rdeg_ref, ws_ref, wn_ref, b_ref, ga_ref,
                 be_ref, out_ref):
  mean = (p_ref[0, :N] + p_ref[1, :N]) * rdeg_ref[...]
  z = (jnp.dot(h_ref[...], ws_ref[...], preferred_element_type=jnp.float32)
       + jnp.dot(mean, wn_ref[...], preferred_element_type=jnp.float32)
       + b_ref[...])
  mu = jnp.mean(z, axis=0, keepdims=True)
  var = jnp.mean((z - mu) * (z - mu), axis=0, keepdims=True)
  zn = ga_ref[...] * (z - mu) * lax.rsqrt(var + EPS) + be_ref[...]
  out_ref[...] = jnp.maximum(zn, 0.0)


_tc_mid = pl.pallas_call(
    _tc_mid_body,
    out_shape=jax.ShapeDtypeStruct((N, D), jnp.float32))


def _tc_last_body(h_ref, p_ref, rdeg_ref, ws_ref, wn_ref, b_ref, out_ref):
  mean = (p_ref[0, :N] + p_ref[1, :N]) * rdeg_ref[...]
  out_ref[...] = (
      jnp.dot(h_ref[...], ws_ref[...], preferred_element_type=jnp.float32)
      + jnp.dot(mean, wn_ref[...], preferred_element_type=jnp.float32)
      + b_ref[...])


_tc_last = pl.pallas_call(
    _tc_last_body,
    out_shape=jax.ShapeDtypeStruct((N, D), jnp.float32))


def kernel(x, edge_index, W_self0, W_neigh0, b0, W_self1, W_neigh1, b1,
           W_self2, W_neigh2, b2, gamma0, beta0, gamma1, beta1):
  src = edge_index[0].astype(jnp.int32)
  dst = edge_index[1].astype(jnp.int32)
  zrows = jnp.zeros((RPSP, D), jnp.float32)
  onesrow = jnp.ones((C, D), jnp.float32)
  b0r, b1r, b2r = (b.reshape(1, D) for b in (b0, b1, b2))
  g0r, g1r = gamma0.reshape(1, D), gamma1.reshape(1, D)
  be0r, be1r = beta0.reshape(1, D), beta1.reshape(1, D)

  degp = _deg(dst, onesrow, zrows)
  p0 = _agg(x, src, dst, zrows)
  h1, rdeg = _tc0(x, p0, degp, W_self0, W_neigh0, b0r, g0r, be0r)
  p1 = _agg(h1, src, dst, zrows)
  h2 = _tc_mid(h1, p1, rdeg, W_self1, W_neigh1, b1r, g1r, be1r)
  p2 = _agg(h2, src, dst, zrows)
  out = _tc_last(h2, p2, rdeg, W_self2, W_neigh2, b2r)
  return out


# speedup vs baseline: 8.6814x; 1.0831x over previous
import jax
import jax.numpy as jnp
from jax import lax
from jax.experimental import pallas as pl
from jax.experimental.pallas import tpu as pltpu
from jax.experimental.pallas import tpu_sc as plsc

N = 10000
E = 320000
D = 128
EPS = 1e-5
NC = 2
NS = 16
NW = NC * NS
EPW = E // NW
C = 80
NCHUNK = EPW // C
RPSP = 632
NPAD = NS * RPSP

_MESH = plsc.VectorSubcoreMesh(
    core_axis_name="c", subcore_axis_name="s",
    num_cores=NC, num_subcores=NS)


def _agg_body(h_hbm, src_hbm, dst_hbm, zrows_hbm, p_hbm,
              sa, da, sb, db, rows_a, rows_b, acc_sh,
              sem_ia, sem_ib, sem_ga, sem_gb):
  cid = lax.axis_index("c")
  sid = lax.axis_index("s")
  wid = sid * NC + cid
  pltpu.sync_copy(zrows_hbm, acc_sh.at[pl.ds(sid * RPSP, RPSP)])
  plsc.subcore_barrier()
  base = wid * EPW

  def idx_issue(g, s_buf, d_buf, sem):
    off = base + g * C
    pltpu.async_copy(src_hbm.at[pl.ds(off, C)], s_buf, sem)
    pltpu.async_copy(dst_hbm.at[pl.ds(off, C)], d_buf, sem)

  def idx_wait(s_buf, d_buf, sem):
    pltpu.make_async_copy(src_hbm.at[pl.ds(0, C)], s_buf, sem).wait()
    pltpu.make_async_copy(dst_hbm.at[pl.ds(0, C)], d_buf, sem).wait()

  idx_issue(0, sa, da, sem_ia)
  idx_issue(1, sb, db, sem_ib)
  idx_wait(sa, da, sem_ia)
  pltpu.async_copy(h_hbm.at[sa], rows_a, sem_ga)

  def phase(g, s_x, d_x, rows_x, sem_ix, sem_iy, sem_gx, s_y, d_y, rows_y,
            sem_gy):
    pltpu.make_async_copy(h_hbm.at[s_x], rows_x, sem_gx).wait()
    idx_wait(s_y, d_y, sem_iy)
    pltpu.async_copy(h_hbm.at[s_y], rows_y, sem_gy)
    pltpu.sync_copy(rows_x, acc_sh.at[d_x], add=True)
    @pl.when(g + 2 < NCHUNK)
    def _():
      idx_issue(g + 2, s_x, d_x, sem_ix)

  def pair(g2, carry):
    ga = 2 * g2
    phase(ga, sa, da, rows_a, sem_ia, sem_ib, sem_ga, sb, db, rows_b, sem_gb)
    phase(ga + 1, sb, db, rows_b, sem_ib, sem_ia, sem_gb, sa, da, rows_a,
          sem_ga)
    return carry

  lax.fori_loop(0, (NCHUNK - 1) // 2, pair, 0)
  pltpu.make_async_copy(h_hbm.at[sa], rows_a, sem_ga).wait()
  pltpu.sync_copy(rows_a, acc_sh.at[da], add=True)
  plsc.subcore_barrier()
  pltpu.sync_copy(acc_sh.at[pl.ds(sid * RPSP, RPSP)],
                  p_hbm.at[cid, pl.ds(sid * RPSP, RPSP)])


_agg = pl.kernel(
    _agg_body,
    out_type=jax.ShapeDtypeStruct((NC, NPAD, D), jnp.float32),
    mesh=_MESH,
    scratch_types=(
        pltpu.VMEM((C,), jnp.int32),
        pltpu.VMEM((C,), jnp.int32),
        pltpu.VMEM((C,), jnp.int32),
        pltpu.VMEM((C,), jnp.int32),
        pltpu.VMEM((C, D), jnp.float32),
        pltpu.VMEM((C, D), jnp.float32),
        pltpu.VMEM_SHARED((NPAD, D), jnp.float32),
        pltpu.SemaphoreType.DMA,
        pltpu.SemaphoreType.DMA,
        pltpu.SemaphoreType.DMA,
        pltpu.SemaphoreType.DMA,
    ))


def _deg_body(dst_hbm, ones_hbm, zrows_hbm, degp_hbm,
              d_a, d_b, ones_v, dacc_sh, sem_ia, sem_ib, sem_sa, sem_sb):
  cid = lax.axis_index("c")
  sid = lax.axis_index("s")
  wid = sid * NC + cid
  pltpu.sync_copy(zrows_hbm, dacc_sh.at[pl.ds(sid * RPSP, RPSP)])
  pltpu.sync_copy(ones_hbm, ones_v)
  plsc.subcore_barrier()
  base = wid * EPW

  def idx_issue(g, d_buf, sem):
    pltpu.async_copy(dst_hbm.at[pl.ds(base + g * C, C)], d_buf, sem)

  def idx_wait(d_buf, sem):
    pltpu.make_async_copy(dst_hbm.at[pl.ds(0, C)], d_buf, sem).wait()

  def sc_wait(d_buf, sem):
    pltpu.make_async_copy(ones_v, dacc_sh.at[d_buf], sem).wait()

  idx_issue(0, d_a, sem_ia)
  idx_issue(1, d_b, sem_ib)
  idx_wait(d_a, sem_ia)
  pltpu.async_copy(ones_v, dacc_sh.at[d_a], sem_sa, add=True)

  def phase(g, d_x, sem_ix, sem_sx, d_y, sem_iy, sem_sy):
    idx_wait(d_x, sem_ix)
    pltpu.async_copy(ones_v, dacc_sh.at[d_x], sem_sx, add=True)
    sc_wait(d_y, sem_sy)
    @pl.when(g + 1 < NCHUNK)
    def _():
      idx_issue(g + 1, d_y, sem_iy)

  def pair(g2, carry):
    g = 1 + 2 * g2
    phase(g, d_b, sem_ib, sem_sb, d_a, sem_ia, sem_sa)
    phase(g + 1, d_a, sem_ia, sem_sa, d_b, sem_ib, sem_sb)
    return carry

  lax.fori_loop(0, (NCHUNK - 1) // 2, pair, 0)
  sc_wait(d_a, sem_sa)
  plsc.subcore_barrier()
  pltpu.sync_copy(dacc_sh.at[pl.ds(sid * RPSP, RPSP)],
                  degp_hbm.at[cid, pl.ds(sid * RPSP, RPSP)])


_deg = pl.kernel(
    _deg_body,
    out_type=jax.ShapeDtypeStruct((NC, NPAD, D), jnp.float32),
    mesh=_MESH,
    scratch_types=(
        pltpu.VMEM((C,), jnp.int32),
        pltpu.VMEM((C,), jnp.int32),
        pltpu.VMEM((C, D), jnp.float32),
        pltpu.VMEM_SHARED((NPAD, D), jnp.float32),
        pltpu.SemaphoreType.DMA,
        pltpu.SemaphoreType.DMA,
        pltpu.SemaphoreType.DMA,
        pltpu.SemaphoreType.DMA,
    ))


def _tc0_body(x_ref, p_ref, degp_ref, ws_ref, wn_ref, b_ref, ga_ref, be_ref,
              out_ref, rdeg_ref):
  deg = degp_ref[0, :N, :1] + degp_ref[1, :N, :1]
  rdeg = 1.0 / jnp.maximum(deg, 1.0)
  rdeg_ref[...] = rdeg
  mean = (p_ref[0, :N] + p_ref[1, :N]) * rdeg
  z = (jnp.dot(x_ref[...], ws_ref[...], preferred_element_type=jnp.float32)
       + jnp.dot(mean, wn_ref[...], preferred_element_type=jnp.float32)
       + b_ref[...])
  mu = jnp.mean(z, axis=0, keepdims=True)
  var = jnp.mean((z - mu) * (z - mu), axis=0, keepdims=True)
  zn = ga_ref[...] * (z - mu) * lax.rsqrt(var + EPS) + be_ref[...]
  out_ref[...] = jnp.maximum(zn, 0.0)


_tc0 = pl.pallas_call(
    _tc0_body,
    out_shape=(jax.ShapeDtypeStruct((N, D), jnp.float32),
               jax.ShapeDtypeStruct((N, 1), jnp.float32)))


def _tc_mid_body(h_ref, p_ref, rdeg_ref, ws_ref, wn_ref, b_ref, ga_ref,
                 be_ref, out_ref):
  mean = (p_ref[0, :N] + p_ref[1, :N]) * rdeg_ref[...]
  z = (jnp.dot(h_ref[...], ws_ref[...], preferred_element_type=jnp.float32)
       + jnp.dot(mean, wn_ref[...], preferred_element_type=jnp.float32)
       + b_ref[...])
  mu = jnp.mean(z, axis=0, keepdims=True)
  var = jnp.mean((z - mu) * (z - mu), axis=0, keepdims=True)
  zn = ga_ref[...] * (z - mu) * lax.rsqrt(var + EPS) + be_ref[...]
  out_ref[...] = jnp.maximum(zn, 0.0)


_tc_mid = pl.pallas_call(
    _tc_mid_body,
    out_shape=jax.ShapeDtypeStruct((N, D), jnp.float32))


def _tc_last_body(h_ref, p_ref, rdeg_ref, ws_ref, wn_ref, b_ref, out_ref):
  mean = (p_ref[0, :N] + p_ref[1, :N]) * rdeg_ref[...]
  out_ref[...] = (
      jnp.dot(h_ref[...], ws_ref[...], preferred_element_type=jnp.float32)
      + jnp.dot(mean, wn_ref[...], preferred_element_type=jnp.float32)
      + b_ref[...])


_tc_last = pl.pallas_call(
    _tc_last_body,
    out_shape=jax.ShapeDtypeStruct((N, D), jnp.float32))


def kernel(x, edge_index, W_self0, W_neigh0, b0, W_self1, W_neigh1, b1,
           W_self2, W_neigh2, b2, gamma0, beta0, gamma1, beta1):
  src = edge_index[0].astype(jnp.int32)
  dst = edge_index[1].astype(jnp.int32)
  zrows = jnp.zeros((RPSP, D), jnp.float32)
  onesrow = jnp.ones((C, D), jnp.float32)
  b0r, b1r, b2r = (b.reshape(1, D) for b in (b0, b1, b2))
  g0r, g1r = gamma0.reshape(1, D), gamma1.reshape(1, D)
  be0r, be1r = beta0.reshape(1, D), beta1.reshape(1, D)

  degp = _deg(dst, onesrow, zrows)
  p0 = _agg(x, src, dst, zrows)
  h1, rdeg = _tc0(x, p0, degp, W_self0, W_neigh0, b0r, g0r, be0r)
  p1 = _agg(h1, src, dst, zrows)
  h2 = _tc_mid(h1, p1, rdeg, W_self1, W_neigh1, b1r, g1r, be1r)
  p2 = _agg(h2, src, dst, zrows)
  out = _tc_last(h2, p2, rdeg, W_self2, W_neigh2, b2r)
  return out


# speedup vs baseline: 8.6816x; 1.0000x over previous
import jax
import jax.numpy as jnp
from jax import lax
from jax.experimental import pallas as pl
from jax.experimental.pallas import tpu as pltpu
from jax.experimental.pallas import tpu_sc as plsc

N = 10000
E = 320000
D = 128
EPS = 1e-5
NC = 2
NS = 16
NW = NC * NS
EPW = E // NW
C = 80
NCHUNK = EPW // C
RPSP = 632
NPAD = NS * RPSP

_MESH = plsc.VectorSubcoreMesh(
    core_axis_name="c", subcore_axis_name="s",
    num_cores=NC, num_subcores=NS)


def _agg_body(h_hbm, src_hbm, dst_hbm, zrows_hbm, p_hbm,
              s0, d0, s1, d1, s2, d2, r0, r1, r2, acc_sh,
              si0, si1, si2, sg0, sg1, sg2, ss0, ss1, ss2):
  cid = lax.axis_index("c")
  sid = lax.axis_index("s")
  wid = sid * NC + cid
  pltpu.sync_copy(zrows_hbm, acc_sh.at[pl.ds(sid * RPSP, RPSP)])
  plsc.subcore_barrier()
  base = wid * EPW

  def idx_issue(g, s_buf, d_buf, sem):
    off = base + g * C
    pltpu.async_copy(src_hbm.at[pl.ds(off, C)], s_buf, sem)
    pltpu.async_copy(dst_hbm.at[pl.ds(off, C)], d_buf, sem)

  def idx_wait(s_buf, d_buf, sem):
    pltpu.make_async_copy(src_hbm.at[pl.ds(0, C)], s_buf, sem).wait()
    pltpu.make_async_copy(dst_hbm.at[pl.ds(0, C)], d_buf, sem).wait()

  def sc_wait(rows, d_buf, sem):
    pltpu.make_async_copy(rows, acc_sh.at[d_buf], sem).wait()

  idx_issue(0, s0, d0, si0)
  idx_issue(1, s1, d1, si1)
  idx_wait(s0, d0, si0)
  pltpu.async_copy(h_hbm.at[s0], r0, sg0)
  pltpu.make_async_copy(h_hbm.at[s0], r0, sg0).wait()
  idx_wait(s1, d1, si1)
  pltpu.async_copy(h_hbm.at[s1], r1, sg1)
  pltpu.async_copy(r0, acc_sh.at[d0], ss0, add=True)
  idx_issue(2, s2, d2, si2)
  pltpu.make_async_copy(h_hbm.at[s1], r1, sg1).wait()
  idx_wait(s2, d2, si2)
  pltpu.async_copy(h_hbm.at[s2], r2, sg2)
  pltpu.async_copy(r1, acc_sh.at[d1], ss1, add=True)
  sc_wait(r0, d0, ss0)
  idx_issue(3, s0, d0, si0)

  def phase(g, s_x, d_x, r_x, si_x, sg_x, ss_x,
            s_y, d_y, r_y, si_y, sg_y,
            s_z, d_z, r_z, si_z, ss_z):
    pltpu.make_async_copy(h_hbm.at[s_x], r_x, sg_x).wait()
    @pl.when(g + 1 < NCHUNK)
    def _():
      idx_wait(s_y, d_y, si_y)
      pltpu.async_copy(h_hbm.at[s_y], r_y, sg_y)
    pltpu.async_copy(r_x, acc_sh.at[d_x], ss_x, add=True)
    sc_wait(r_z, d_z, ss_z)
    @pl.when(g + 2 < NCHUNK)
    def _():
      idx_issue(g + 2, s_z, d_z, si_z)

  b0 = (s0, d0, r0, si0, sg0, ss0)
  b1 = (s1, d1, r1, si1, sg1, ss1)
  b2 = (s2, d2, r2, si2, sg2, ss2)

  def triple(i, carry):
    g = 3 * i + 2
    phase(g, b2[0], b2[1], b2[2], b2[3], b2[4], b2[5],
          b0[0], b0[1], b0[2], b0[3], b0[4],
          b1[0], b1[1], b1[2], b1[3], b1[5])
    phase(g + 1, b0[0], b0[1], b0[2], b0[3], b0[4], b0[5],
          b1[0], b1[1], b1[2], b1[3], b1[4],
          b2[0], b2[1], b2[2], b2[3], b2[5])
    phase(g + 2, b1[0], b1[1], b1[2], b1[3], b1[4], b1[5],
          b2[0], b2[1], b2[2], b2[3], b2[4],
          b0[0], b0[1], b0[2], b0[3], b0[5])
    return carry

  lax.fori_loop(0, (NCHUNK - 2) // 3, triple, 0)
  sc_wait(r1, d1, ss1)
  plsc.subcore_barrier()
  pltpu.sync_copy(acc_sh.at[pl.ds(sid * RPSP, RPSP)],
                  p_hbm.at[cid, pl.ds(sid * RPSP, RPSP)])


_agg = pl.kernel(
    _agg_body,
    out_type=jax.ShapeDtypeStruct((NC, NPAD, D), jnp.float32),
    mesh=_MESH,
    scratch_types=(
        pltpu.VMEM((C,), jnp.int32),
        pltpu.VMEM((C,), jnp.int32),
        pltpu.VMEM((C,), jnp.int32),
        pltpu.VMEM((C,), jnp.int32),
        pltpu.VMEM((C,), jnp.int32),
        pltpu.VMEM((C,), jnp.int32),
        pltpu.VMEM((C, D), jnp.float32),
        pltpu.VMEM((C, D), jnp.float32),
        pltpu.VMEM((C, D), jnp.float32),
        pltpu.VMEM_SHARED((NPAD, D), jnp.float32),
        pltpu.SemaphoreType.DMA,
        pltpu.SemaphoreType.DMA,
        pltpu.SemaphoreType.DMA,
        pltpu.SemaphoreType.DMA,
        pltpu.SemaphoreType.DMA,
        pltpu.SemaphoreType.DMA,
        pltpu.SemaphoreType.DMA,
        pltpu.SemaphoreType.DMA,
        pltpu.SemaphoreType.DMA,
    ))


def _deg_body(dst_hbm, ones_hbm, zrows_hbm, degp_hbm,
              d_a, d_b, ones_v, dacc_sh, sem_ia, sem_ib, sem_sa, sem_sb):
  cid = lax.axis_index("c")
  sid = lax.axis_index("s")
  wid = sid * NC + cid
  pltpu.sync_copy(zrows_hbm, dacc_sh.at[pl.ds(sid * RPSP, RPSP)])
  pltpu.sync_copy(ones_hbm, ones_v)
  plsc.subcore_barrier()
  base = wid * EPW

  def idx_issue(g, d_buf, sem):
    pltpu.async_copy(dst_hbm.at[pl.ds(base + g * C, C)], d_buf, sem)

  def idx_wait(d_buf, sem):
    pltpu.make_async_copy(dst_hbm.at[pl.ds(0, C)], d_buf, sem).wait()

  def sc_wait(d_buf, sem):
    pltpu.make_async_copy(ones_v, dacc_sh.at[d_buf], sem).wait()

  idx_issue(0, d_a, sem_ia)
  idx_issue(1, d_b, sem_ib)
  idx_wait(d_a, sem_ia)
  pltpu.async_copy(ones_v, dacc_sh.at[d_a], sem_sa, add=True)

  def phase(g, d_x, sem_ix, sem_sx, d_y, sem_iy, sem_sy):
    idx_wait(d_x, sem_ix)
    pltpu.async_copy(ones_v, dacc_sh.at[d_x], sem_sx, add=True)
    sc_wait(d_y, sem_sy)
    @pl.when(g + 1 < NCHUNK)
    def _():
      idx_issue(g + 1, d_y, sem_iy)

  def pair(g2, carry):
    g = 1 + 2 * g2
    phase(g, d_b, sem_ib, sem_sb, d_a, sem_ia, sem_sa)
    phase(g + 1, d_a, sem_ia, sem_sa, d_b, sem_ib, sem_sb)
    return carry

  lax.fori_loop(0, (NCHUNK - 1) // 2, pair, 0)
  sc_wait(d_a, sem_sa)
  plsc.subcore_barrier()
  pltpu.sync_copy(dacc_sh.at[pl.ds(sid * RPSP, RPSP)],
                  degp_hbm.at[cid, pl.ds(sid * RPSP, RPSP)])


_deg = pl.kernel(
    _deg_body,
    out_type=jax.ShapeDtypeStruct((NC, NPAD, D), jnp.float32),
    mesh=_MESH,
    scratch_types=(
        pltpu.VMEM((C,), jnp.int32),
        pltpu.VMEM((C,), jnp.int32),
        pltpu.VMEM((C, D), jnp.float32),
        pltpu.VMEM_SHARED((NPAD, D), jnp.float32),
        pltpu.SemaphoreType.DMA,
        pltpu.SemaphoreType.DMA,
        pltpu.SemaphoreType.DMA,
        pltpu.SemaphoreType.DMA,
    ))


def _tc0_body(x_ref, p_ref, degp_ref, ws_ref, wn_ref, b_ref, ga_ref, be_ref,
              out_ref, rdeg_ref):
  deg = degp_ref[0, :N, :1] + degp_ref[1, :N, :1]
  rdeg = 1.0 / jnp.maximum(deg, 1.0)
  rdeg_ref[...] = rdeg
  mean = (p_ref[0, :N] + p_ref[1, :N]) * rdeg
  z = (jnp.dot(x_ref[...], ws_ref[...], preferred_element_type=jnp.float32)
       + jnp.dot(mean, wn_ref[...], preferred_element_type=jnp.float32)
       + b_ref[...])
  mu = jnp.mean(z, axis=0, keepdims=True)
  var = jnp.mean((z - mu) * (z - mu), axis=0, keepdims=True)
  zn = ga_ref[...] * (z - mu) * lax.rsqrt(var + EPS) + be_ref[...]
  out_ref[...] = jnp.maximum(zn, 0.0)


_tc0 = pl.pallas_call(
    _tc0_body,
    out_shape=(jax.ShapeDtypeStruct((N, D), jnp.float32),
               jax.ShapeDtypeStruct((N, 1), jnp.float32)))


def _tc_mid_body(h_ref, p_ref, rdeg_ref, ws_ref, wn_ref, b_ref, ga_ref,
                 be_ref, out_ref):
  mean = (p_ref[0, :N] + p_ref[1, :N]) * rdeg_ref[...]
  z = (jnp.dot(h_ref[...], ws_ref[...], preferred_element_type=jnp.float32)
       + jnp.dot(mean, wn_ref[...], preferred_element_type=jnp.float32)
       + b_ref[...])
  mu = jnp.mean(z, axis=0, keepdims=True)
  var = jnp.mean((z - mu) * (z - mu), axis=0, keepdims=True)
  zn = ga_ref[...] * (z - mu) * lax.rsqrt(var + EPS) + be_ref[...]
  out_ref[...] = jnp.maximum(zn, 0.0)


_tc_mid = pl.pallas_call(
    _tc_mid_body,
    out_shape=jax.ShapeDtypeStruct((N, D), jnp.float32))


def _tc_last_body(h_ref, p_ref, rdeg_ref, ws_ref, wn_ref, b_ref, out_ref):
  mean = (p_ref[0, :N] + p_ref[1, :N]) * rdeg_ref[...]
  out_ref[...] = (
      jnp.dot(h_ref[...], ws_ref[...], preferred_element_type=jnp.float32)
      + jnp.dot(mean, wn_ref[...], preferred_element_type=jnp.float32)
      + b_ref[...])


_tc_last = pl.pallas_call(
    _tc_last_body,
    out_shape=jax.ShapeDtypeStruct((N, D), jnp.float32))


def kernel(x, edge_index, W_self0, W_neigh0, b0, W_self1, W_neigh1, b1,
           W_self2, W_neigh2, b2, gamma0, beta0, gamma1, beta1):
  src = edge_index[0].astype(jnp.int32)
  dst = edge_index[1].astype(jnp.int32)
  zrows = jnp.zeros((RPSP, D), jnp.float32)
  onesrow = jnp.ones((C, D), jnp.float32)
  b0r, b1r, b2r = (b.reshape(1, D) for b in (b0, b1, b2))
  g0r, g1r = gamma0.reshape(1, D), gamma1.reshape(1, D)
  be0r, be1r = beta0.reshape(1, D), beta1.reshape(1, D)

  degp = _deg(dst, onesrow, zrows)
  p0 = _agg(x, src, dst, zrows)
  h1, rdeg = _tc0(x, p0, degp, W_self0, W_neigh0, b0r, g0r, be0r)
  p1 = _agg(h1, src, dst, zrows)
  h2 = _tc_mid(h1, p1, rdeg, W_self1, W_neigh1, b1r, g1r, be1r)
  p2 = _agg(h2, src, dst, zrows)
  out = _tc_last(h2, p2, rdeg, W_self2, W_neigh2, b2r)
  return out
